# baseline (speedup 1.0000x reference)
import jax
import jax.numpy as jnp
from jax.experimental import pallas as pl
from jax.experimental.pallas import tpu as pltpu

_BM = 200


def _outer(x_ref, l_hbm, u_hbm, wi_ref, ws_ref, out_hbm, xb_ref, *, bm):
    n, d = x_ref.shape
    d_out = wi_ref.shape[1]
    xb_ref[...] = x_ref[...].astype(jnp.bfloat16)

    def inner(l_ref, u_ref, o_ref):
        lb = l_ref[...].astype(jnp.bfloat16)
        ub = u_ref[...].astype(jnp.bfloat16)
        t_l = jnp.dot(lb, xb_ref[...], preferred_element_type=jnp.float32)
        t_u = jnp.dot(ub, xb_ref[...], preferred_element_type=jnp.float32)
        t = (jnp.dot(t_l, wi_ref[...], preferred_element_type=jnp.float32)
             + jnp.dot(t_u, ws_ref[...], preferred_element_type=jnp.float32))
        o_ref[...] = jnp.maximum(t, 0.0)

    pipe = pltpu.emit_pipeline(
        inner,
        grid=(n // bm,),
        in_specs=[
            pl.BlockSpec((bm, n), lambda m: (m, 0),
                         pipeline_mode=pl.Buffered(buffer_count=3)),
            pl.BlockSpec((bm, n), lambda m: (m, 0),
                         pipeline_mode=pl.Buffered(buffer_count=3)),
        ],
        out_specs=[pl.BlockSpec((bm, d_out), lambda m: (m, 0))],
    )
    pipe(l_hbm, u_hbm, out_hbm)


def _run(x, lower, upper, w_irr, w_sol, bm):
    import functools
    n, d = x.shape
    d_out = w_irr.shape[1]
    return pl.pallas_call(
        functools.partial(_outer, bm=bm),
        in_specs=[
            pl.BlockSpec(memory_space=pltpu.MemorySpace.VMEM),
            pl.BlockSpec(memory_space=pltpu.MemorySpace.HBM),
            pl.BlockSpec(memory_space=pltpu.MemorySpace.HBM),
            pl.BlockSpec(memory_space=pltpu.MemorySpace.VMEM),
            pl.BlockSpec(memory_space=pltpu.MemorySpace.VMEM),
        ],
        out_specs=pl.BlockSpec(memory_space=pltpu.MemorySpace.HBM),
        out_shape=jax.ShapeDtypeStruct((n, d_out), jnp.float32),
        scratch_shapes=[pltpu.VMEM((n, d), jnp.bfloat16)],
    )(x, lower, upper, w_irr, w_sol)


def kernel(x, lower_neighborhood, upper_neighborhood, W_irr, W_sol):
    return _run(x, lower_neighborhood, upper_neighborhood, W_irr, W_sol, _BM)
